# baseline (speedup 1.0000x reference)
import jax
import jax.numpy as jnp
from jax.experimental import pallas as pl

N = 32768
D = 64
FF = 48
H = 3 * FF

BN = 4096


def _moe_block(x_ref, w1a_ref, b1a_ref, w1b_ref, b1b_ref,
               sw2_ref, sb2_ref, rw2_ref, rb2_ref, gw_ref, gb_ref,
               out_ref):
    x = x_ref[...]

    ha = jnp.dot(x, w1a_ref[...], preferred_element_type=jnp.float32) + b1a_ref[...]
    hb = jnp.dot(x, w1b_ref[...], preferred_element_type=jnp.float32) + b1b_ref[...]
    act = (ha * jax.nn.sigmoid(ha)) * hb

    shared = jnp.dot(act[:, :FF], sw2_ref[...],
                     preferred_element_type=jnp.float32) + sb2_ref[...]
    o0 = jnp.dot(act[:, FF:2 * FF], rw2_ref[0],
                 preferred_element_type=jnp.float32) + rb2_ref[0]
    o1 = jnp.dot(act[:, 2 * FF:], rw2_ref[1],
                 preferred_element_type=jnp.float32) + rb2_ref[1]

    logits = jnp.dot(x, gw_ref[...], preferred_element_type=jnp.float32) + gb_ref[...]
    l0 = logits[:, 0:1]
    l1 = logits[:, 1:2]
    pick1 = l1 > l0
    w = jax.nn.sigmoid(jnp.abs(l1 - l0))
    routed = jnp.where(pick1, o1, o0) * w
    out_ref[...] = shared + routed


@jax.jit
def kernel(x, sw1, sb1, sw2, sb2, rw1, rb1, rw2, rb2, gw, gb):
    w1a = jnp.concatenate([sw1[:, :FF], rw1[0][:, :FF], rw1[1][:, :FF]], axis=1)
    w1b = jnp.concatenate([sw1[:, FF:], rw1[0][:, FF:], rw1[1][:, FF:]], axis=1)
    b1a = jnp.concatenate([sb1[:FF], rb1[0][:FF], rb1[1][:FF]], axis=0)[None]
    b1b = jnp.concatenate([sb1[FF:], rb1[0][FF:], rb1[1][FF:]], axis=0)[None]

    grid = (N // BN,)
    full = lambda *s: pl.BlockSpec(s, lambda i: (0,) * len(s))
    return pl.pallas_call(
        _moe_block,
        grid=grid,
        in_specs=[
            pl.BlockSpec((BN, D), lambda i: (i, 0)),
            full(D, H), full(1, H), full(D, H), full(1, H),
            full(FF, D), full(D),
            full(2, FF, D), full(2, D),
            full(D, 2), full(2),
        ],
        out_specs=pl.BlockSpec((BN, D), lambda i: (i, 0)),
        out_shape=jax.ShapeDtypeStruct((N, D), jnp.float32),
    )(x, w1a, b1a, w1b, b1b, sw2, sb2, rw2, rb2, gw, gb)


# speedup vs baseline: 1.1712x; 1.1712x over previous
import jax
import jax.numpy as jnp
from jax.experimental import pallas as pl

N = 32768
D = 64
FF = 48

BN = 4096


def _moe_block(x_ref, sw1_ref, sb1_ref, sw2_ref, sb2_ref,
               rw1_ref, rb1_ref, rw2_ref, rb2_ref, gw_ref, gb_ref,
               out_ref):
    x = x_ref[...]

    def expert(w1, b1, w2, b2):
        h = jnp.dot(x, w1, preferred_element_type=jnp.float32) + b1
        a = h[:, :FF]
        b = h[:, FF:]
        act = (a * jax.nn.sigmoid(a)) * b
        return jnp.dot(act, w2, preferred_element_type=jnp.float32) + b2

    shared = expert(sw1_ref[...], sb1_ref[...], sw2_ref[...], sb2_ref[...])
    o0 = expert(rw1_ref[0], rb1_ref[0], rw2_ref[0], rb2_ref[0])
    o1 = expert(rw1_ref[1], rb1_ref[1], rw2_ref[1], rb2_ref[1])

    l0 = jnp.dot(x, gw_ref[0], preferred_element_type=jnp.float32) + gb_ref[0, 0]
    l1 = jnp.dot(x, gw_ref[1], preferred_element_type=jnp.float32) + gb_ref[1, 0]
    pick1 = l1 > l0
    w = jax.nn.sigmoid(jnp.abs(l1 - l0))
    routed = jnp.where(pick1, o1, o0) * w
    out_ref[...] = shared + routed


@jax.jit
def kernel(x, sw1, sb1, sw2, sb2, rw1, rb1, rw2, rb2, gw, gb):
    gwrep = jnp.stack([jnp.tile(gw[:, 0:1], (1, D)),
                       jnp.tile(gw[:, 1:2], (1, D))])
    gbrep = gb[:, None]
    grid = (N // BN,)
    full = lambda *s: pl.BlockSpec(s, lambda i: (0,) * len(s))
    return pl.pallas_call(
        _moe_block,
        grid=grid,
        in_specs=[
            pl.BlockSpec((BN, D), lambda i: (i, 0)),
            full(D, 2 * FF), full(2 * FF), full(FF, D), full(D),
            full(2, D, 2 * FF), full(2, 2 * FF), full(2, FF, D), full(2, D),
            full(2, D, D), full(2, 1),
        ],
        out_specs=pl.BlockSpec((BN, D), lambda i: (i, 0)),
        out_shape=jax.ShapeDtypeStruct((N, D), jnp.float32),
    )(x, sw1, sb1, sw2, sb2, rw1, rb1, rw2, rb2, gwrep, gbrep)


# speedup vs baseline: 1.1984x; 1.0232x over previous
import jax
import jax.numpy as jnp
from jax.experimental import pallas as pl

N = 32768
D = 64
FF = 48

BN = 8192


def _moe_block(x_ref, sw1_ref, sb1_ref, sw2_ref, sb2_ref,
               rw1_ref, rb1_ref, rw2_ref, rb2_ref, gw_ref, gb_ref,
               out_ref):
    x = x_ref[...]

    def expert(w1, b1, w2, b2):
        h = jnp.dot(x, w1, preferred_element_type=jnp.float32) + b1
        a = h[:, :FF]
        b = h[:, FF:]
        act = (a * jax.nn.sigmoid(a)) * b
        return jnp.dot(act, w2, preferred_element_type=jnp.float32) + b2

    shared = expert(sw1_ref[...], sb1_ref[...], sw2_ref[...], sb2_ref[...])
    o0 = expert(rw1_ref[0], rb1_ref[0], rw2_ref[0], rb2_ref[0])
    o1 = expert(rw1_ref[1], rb1_ref[1], rw2_ref[1], rb2_ref[1])

    l0 = jnp.dot(x, gw_ref[0], preferred_element_type=jnp.float32) + gb_ref[0, 0]
    l1 = jnp.dot(x, gw_ref[1], preferred_element_type=jnp.float32) + gb_ref[1, 0]
    pick1 = l1 > l0
    w = jax.nn.sigmoid(jnp.abs(l1 - l0))
    routed = jnp.where(pick1, o1, o0) * w
    out_ref[...] = shared + routed


@jax.jit
def kernel(x, sw1, sb1, sw2, sb2, rw1, rb1, rw2, rb2, gw, gb):
    gwrep = jnp.stack([jnp.tile(gw[:, 0:1], (1, D)),
                       jnp.tile(gw[:, 1:2], (1, D))])
    gbrep = gb[:, None]
    grid = (N // BN,)
    full = lambda *s: pl.BlockSpec(s, lambda i: (0,) * len(s))
    return pl.pallas_call(
        _moe_block,
        grid=grid,
        in_specs=[
            pl.BlockSpec((BN, D), lambda i: (i, 0)),
            full(D, 2 * FF), full(2 * FF), full(FF, D), full(D),
            full(2, D, 2 * FF), full(2, 2 * FF), full(2, FF, D), full(2, D),
            full(2, D, D), full(2, 1),
        ],
        out_specs=pl.BlockSpec((BN, D), lambda i: (i, 0)),
        out_shape=jax.ShapeDtypeStruct((N, D), jnp.float32),
    )(x, sw1, sb1, sw2, sb2, rw1, rb1, rw2, rb2, gwrep, gbrep)
